# baseline (speedup 1.0000x reference)
import jax
import jax.numpy as jnp
from jax import lax
from jax.experimental import pallas as pl
from jax.experimental.pallas import tpu as pltpu
from jax.experimental.pallas import tpu_sc as plsc

_N_ENT = 10000
_N_ITEMS = 8000
_CH = 128
_HC = 64
_N_EDGES = 320000
_NC = 2
_NS = 16
_K = 128
_G = 160
_EPT = _K * _G
_E_PAD = _NS * _EPT
_ACC_ROWS = 10240
_ROWS_PER_TILE = _ACC_ROWS // _NS
_GARBAGE_ROW = _N_ENT
_NBUF = 5
_PRE = 2


def _edges_body(ei_ref, et_ref, comb_ref, head_ref):
    head = ei_ref[0]
    tail = ei_ref[1]
    rel = jnp.remainder(et_ref[...] - 2, 10)
    base = rel * _N_ENT + tail
    npad = (_E_PAD - _N_EDGES) // _K
    iota = (lax.broadcasted_iota(jnp.int32, (npad, _K), 0) * _K
            + lax.broadcasted_iota(jnp.int32, (npad, _K), 1))
    comb0 = jnp.concatenate([base, iota % _N_ENT], axis=0)
    comb_ref[0] = comb0
    comb_ref[1] = comb0 + 10 * _N_ENT
    head_ref[...] = jnp.concatenate(
        [head, _GARBAGE_ROW + iota % (_ACC_ROWS - _N_ENT)], axis=0)


def _scale_body(ent_ref, w_ref, out_ref):
    c = pl.program_id(1)
    r = pl.program_id(2)
    w = w_ref[pl.ds(r, 1), :]
    x = ent_ref[...] * w
    out_ref[...] = jnp.where(c == 0, x[:, :_HC], x[:, _HC:])


def _norm_body(p0_ref, p1_ref, out_ref):
    x = jnp.concatenate([p0_ref[...], p1_ref[...]], axis=1)
    n = jnp.sqrt(jnp.sum(x * x, axis=1, keepdims=True))
    out_ref[...] = x / jnp.maximum(n, 1e-12)


def _final_body(ent_ref, e1_ref, p0_ref, p1_ref, item_ref, ia_ref, asp_ref,
                w1_ref, w2_ref, out_ref):
    x = jnp.concatenate([p0_ref[...], p1_ref[...]], axis=1)
    n = jnp.sqrt(jnp.sum(x * x, axis=1, keepdims=True))
    e2 = x / jnp.maximum(n, 1e-12)
    ents = ent_ref[...] + e1_ref[...] + e2
    agg = jnp.dot(ia_ref[...], asp_ref[...], preferred_element_type=jnp.float32)
    an = jnp.sqrt(jnp.sum(agg * agg, axis=1, keepdims=True))
    item_part = item_ref[...] + 2.0 * (agg / jnp.maximum(an, 1e-12))
    out_ref[...] = w1_ref[0, 0] * ents + w2_ref[0, 0] * item_part


def _user_body(user_ref, ua_ref, asp_ref, out_ref):
    agg = jnp.dot(ua_ref[...], asp_ref[...], preferred_element_type=jnp.float32)
    n = jnp.sqrt(jnp.sum(agg * agg, axis=1, keepdims=True))
    out_ref[...] = user_ref[...] + 2.0 * (agg / jnp.maximum(n, 1e-12))


def _build_table(ent, weight):
    return pl.pallas_call(
        _scale_body,
        grid=(10, 2, 10),
        in_specs=[
            pl.BlockSpec((1000, _CH), lambda b, c, r: (b, 0)),
            pl.BlockSpec((10, _CH), lambda b, c, r: (0, 0)),
        ],
        out_specs=pl.BlockSpec(
            (1000, _HC), lambda b, c, r: (c * 100 + r * 10 + b, 0)),
        out_shape=jax.ShapeDtypeStruct((2 * 10 * _N_ENT, _HC), jnp.float32),
    )(ent, weight)


def _gather(table, comb_v, rows_v, gsems, g, b):
    return pltpu.async_copy(table.at[comb_v.at[g]], rows_v.at[b], gsems[b])


def _gwait(table, comb_v, rows_v, gsems, g, b):
    pltpu.make_async_copy(table.at[comb_v.at[g]], rows_v.at[b], gsems[b]).wait()


def _scatter(acc, head_v, rows_v, ssems, g, b):
    return pltpu.async_copy(rows_v.at[b], acc.at[head_v.at[g]], ssems[b],
                            add=True)


def _swait(acc, head_v, rows_v, ssems, g, b):
    pltpu.make_async_copy(rows_v.at[b], acc.at[head_v.at[g]], ssems[b]).wait()


def _hop_body(table, comb, head, out, comb_v, head_v, rows_v, acc, *sems):
    gsems, ssems = sems[:_NBUF], sems[_NBUF:]
    cid = lax.axis_index("c")
    sid = lax.axis_index("s")

    pltpu.sync_copy(comb.at[cid, sid], comb_v)
    pltpu.sync_copy(head.at[sid], head_v)

    def _zrow(i, c):
        for j in range(_HC // 16):
            rows_v[0, i, pl.ds(j * 16, 16)] = jnp.zeros((16,), jnp.float32)
        return c
    lax.fori_loop(0, _K, _zrow, 0)
    for t in range(_ROWS_PER_TILE // _K):
        pltpu.sync_copy(
            rows_v.at[0],
            acc.at[pl.ds(sid * _ROWS_PER_TILE + t * _K, _K)])
    plsc.subcore_barrier()

    for b in range(_PRE):
        _gather(table, comb_v, rows_v, gsems, b, b)
    for j in range(_NBUF - _PRE):
        pb = _PRE + j
        _gather(table, comb_v, rows_v, gsems, pb, pb)
        _gwait(table, comb_v, rows_v, gsems, j, j)
        _scatter(acc, head_v, rows_v, ssems, j, j)

    def _step(i, c):
        g0 = (_NBUF - _PRE) + i * _NBUF
        for j in range(_NBUF):
            g = g0 + j
            b = (_NBUF - _PRE + j) % _NBUF
            pb = j
            _swait(acc, head_v, rows_v, ssems, g + _PRE - _NBUF, pb)
            _gather(table, comb_v, rows_v, gsems, g + _PRE, pb)
            _gwait(table, comb_v, rows_v, gsems, g, b)
            _scatter(acc, head_v, rows_v, ssems, g, b)
        return c
    lax.fori_loop(0, (_G - _NBUF) // _NBUF, _step, 0)

    for j in range(_PRE):
        g = _G - _PRE + j
        b = g % _NBUF
        _gwait(table, comb_v, rows_v, gsems, g, b)
        _scatter(acc, head_v, rows_v, ssems, g, b)
    for b in range(_NBUF):
        gg = _G - _NBUF + b
        _swait(acc, head_v, rows_v, ssems, gg, b)

    plsc.subcore_barrier()
    pltpu.sync_copy(
        acc.at[pl.ds(sid * _ROWS_PER_TILE, _ROWS_PER_TILE)],
        out.at[cid, pl.ds(sid * _ROWS_PER_TILE, _ROWS_PER_TILE)])


_hop_cache = []


def _hop(table, comb4, head3):
    if not _hop_cache:
        _hop_cache.append(pl.kernel(
            _hop_body,
            out_type=jax.ShapeDtypeStruct((_NC, _ACC_ROWS, _HC), jnp.float32),
            mesh=plsc.VectorSubcoreMesh(core_axis_name="c",
                                        subcore_axis_name="s"),
            compiler_params=pltpu.CompilerParams(use_tc_tiling_on_sc=False),
            scratch_types=(
                [pltpu.VMEM((_G, _K), jnp.int32),
                 pltpu.VMEM((_G, _K), jnp.int32),
                 pltpu.VMEM((_NBUF, _K, _HC), jnp.float32),
                 pltpu.VMEM_SHARED((_ACC_ROWS, _HC), jnp.float32)]
                + [pltpu.SemaphoreType.DMA] * (2 * _NBUF)),
        ))
    p = _hop_cache[0](table, comb4, head3)
    return p[0], p[1]


def kernel(user_emb, item_emb, entity_emb, aspect_emb, edge_index, edge_type,
           ua_interact_mat, ia_interact_mat, weight, W1, W2):
    f32 = jnp.float32
    et = edge_type.astype(jnp.int32)

    user_res = pl.pallas_call(
        _user_body,
        grid=(50,),
        in_specs=[
            pl.BlockSpec((2000, _CH), lambda b: (b, 0)),
            pl.BlockSpec((2000, 64), lambda b: (b, 0)),
            pl.BlockSpec((64, _CH), lambda b: (0, 0)),
        ],
        out_specs=pl.BlockSpec((2000, _CH), lambda b: (b, 0)),
        out_shape=jax.ShapeDtypeStruct((user_emb.shape[0], _CH), f32),
    )(user_emb, ua_interact_mat, aspect_emb)

    nrow = _N_EDGES // _K
    prow = _E_PAD // _K
    comb, headp = pl.pallas_call(
        _edges_body,
        in_specs=[
            pl.BlockSpec((2, nrow, _K), lambda: (0, 0, 0)),
            pl.BlockSpec((nrow, _K), lambda: (0, 0)),
        ],
        out_specs=[
            pl.BlockSpec((2, prow, _K), lambda: (0, 0, 0)),
            pl.BlockSpec((prow, _K), lambda: (0, 0)),
        ],
        out_shape=[
            jax.ShapeDtypeStruct((2, prow, _K), jnp.int32),
            jax.ShapeDtypeStruct((prow, _K), jnp.int32),
        ],
    )(edge_index.astype(jnp.int32).reshape(2, nrow, _K), et.reshape(nrow, _K))
    comb4 = comb.reshape(_NC, _NS, _G, _K)
    head3 = headp.reshape(_NS, _G, _K)

    table1 = _build_table(entity_emb, weight)
    p1a, p1b = _hop(table1, comb4, head3)
    e1 = pl.pallas_call(
        _norm_body,
        grid=(10,),
        in_specs=[pl.BlockSpec((1000, _HC), lambda b: (b, 0))] * 2,
        out_specs=pl.BlockSpec((1000, _CH), lambda b: (b, 0)),
        out_shape=jax.ShapeDtypeStruct((_N_ENT, _CH), f32),
    )(p1a, p1b)

    table2 = _build_table(e1, weight)
    p2a, p2b = _hop(table2, comb4, head3)

    item_res = pl.pallas_call(
        _final_body,
        grid=(8,),
        in_specs=[
            pl.BlockSpec((1000, _CH), lambda b: (b, 0)),
            pl.BlockSpec((1000, _CH), lambda b: (b, 0)),
            pl.BlockSpec((1000, _HC), lambda b: (b, 0)),
            pl.BlockSpec((1000, _HC), lambda b: (b, 0)),
            pl.BlockSpec((1000, _CH), lambda b: (b, 0)),
            pl.BlockSpec((1000, 64), lambda b: (b, 0)),
            pl.BlockSpec((64, _CH), lambda b: (0, 0)),
            pl.BlockSpec((1, 1), lambda b: (0, 0)),
            pl.BlockSpec((1, 1), lambda b: (0, 0)),
        ],
        out_specs=pl.BlockSpec((1000, _CH), lambda b: (b, 0)),
        out_shape=jax.ShapeDtypeStruct((_N_ITEMS, _CH), f32),
    )(entity_emb, e1, p2a, p2b, item_emb, ia_interact_mat, aspect_emb,
      W1.reshape(1, 1), W2.reshape(1, 1))

    return (item_res, user_res)


# speedup vs baseline: 1.0040x; 1.0040x over previous
import jax
import jax.numpy as jnp
from jax import lax
from jax.experimental import pallas as pl
from jax.experimental.pallas import tpu as pltpu
from jax.experimental.pallas import tpu_sc as plsc

_N_ENT = 10000
_N_ITEMS = 8000
_CH = 128
_HC = 64
_N_EDGES = 320000
_NC = 2
_NS = 16
_K = 128
_G = 160
_EPT = _K * _G
_E_PAD = _NS * _EPT
_ACC_ROWS = 10240
_ROWS_PER_TILE = _ACC_ROWS // _NS
_GARBAGE_ROW = _N_ENT
_NBUF = 5
_PRE = 2


def _edges_body(ei_ref, et_ref, comb_ref, head_ref):
    head = ei_ref[0]
    tail = ei_ref[1]
    rel = jnp.remainder(et_ref[...] - 2, 10)
    base = rel * _N_ENT + tail
    npad = (_E_PAD - _N_EDGES) // _K
    iota = (lax.broadcasted_iota(jnp.int32, (npad, _K), 0) * _K
            + lax.broadcasted_iota(jnp.int32, (npad, _K), 1))
    comb0 = jnp.concatenate([base, iota % _N_ENT], axis=0)
    comb_ref[0] = comb0
    comb_ref[1] = comb0 + 10 * _N_ENT
    head_ref[...] = jnp.concatenate(
        [head, _GARBAGE_ROW + iota % (_ACC_ROWS - _N_ENT)], axis=0)


def _scale_body(ent_ref, w_ref, out_ref):
    c = pl.program_id(1)
    r = pl.program_id(2)
    w = w_ref[pl.ds(r, 1), :]
    x = ent_ref[...] * w
    out_ref[...] = jnp.where(c == 0, x[:, :_HC], x[:, _HC:])


def _l2norm(x):
    n = jnp.sqrt(jnp.sum(x * x, axis=1, keepdims=True))
    return x / jnp.maximum(n, 1e-12)


def _scale_norm_body(p0_ref, p1_ref, w_ref, out_ref, e1_ref):
    c = pl.program_id(1)
    r = pl.program_id(2)

    @pl.when((c == 0) & (r == 0))
    def _():
        e1_ref[...] = _l2norm(
            jnp.concatenate([p0_ref[...], p1_ref[...]], axis=1))

    w = w_ref[pl.ds(r, 1), :]
    x = e1_ref[...] * w
    out_ref[...] = jnp.where(c == 0, x[:, :_HC], x[:, _HC:])


def _final_body(ent_ref, q0_ref, q1_ref, p0_ref, p1_ref, item_ref, ia_ref,
                asp_ref, w1_ref, w2_ref, out_ref):
    e1 = _l2norm(jnp.concatenate([q0_ref[...], q1_ref[...]], axis=1))
    e2 = _l2norm(jnp.concatenate([p0_ref[...], p1_ref[...]], axis=1))
    ents = ent_ref[...] + e1 + e2
    agg = jnp.dot(ia_ref[...], asp_ref[...], preferred_element_type=jnp.float32)
    an = jnp.sqrt(jnp.sum(agg * agg, axis=1, keepdims=True))
    item_part = item_ref[...] + 2.0 * (agg / jnp.maximum(an, 1e-12))
    out_ref[...] = w1_ref[0, 0] * ents + w2_ref[0, 0] * item_part


def _user_body(user_ref, ua_ref, asp_ref, out_ref):
    agg = jnp.dot(ua_ref[...], asp_ref[...], preferred_element_type=jnp.float32)
    n = jnp.sqrt(jnp.sum(agg * agg, axis=1, keepdims=True))
    out_ref[...] = user_ref[...] + 2.0 * (agg / jnp.maximum(n, 1e-12))


def _build_table(ent, weight):
    return pl.pallas_call(
        _scale_body,
        grid=(10, 2, 10),
        in_specs=[
            pl.BlockSpec((1000, _CH), lambda b, c, r: (b, 0)),
            pl.BlockSpec((10, _CH), lambda b, c, r: (0, 0)),
        ],
        out_specs=pl.BlockSpec(
            (1000, _HC), lambda b, c, r: (c * 100 + r * 10 + b, 0)),
        out_shape=jax.ShapeDtypeStruct((2 * 10 * _N_ENT, _HC), jnp.float32),
    )(ent, weight)


def _gather(table, comb_v, rows_v, gsems, g, b):
    return pltpu.async_copy(table.at[comb_v.at[g]], rows_v.at[b], gsems[b])


def _gwait(table, comb_v, rows_v, gsems, g, b):
    pltpu.make_async_copy(table.at[comb_v.at[g]], rows_v.at[b], gsems[b]).wait()


def _scatter(acc, head_v, rows_v, ssems, g, b):
    return pltpu.async_copy(rows_v.at[b], acc.at[head_v.at[g]], ssems[b],
                            add=True)


def _swait(acc, head_v, rows_v, ssems, g, b):
    pltpu.make_async_copy(rows_v.at[b], acc.at[head_v.at[g]], ssems[b]).wait()


def _hop_body(table, comb, head, out, comb_v, head_v, rows_v, acc, *sems):
    gsems, ssems = sems[:_NBUF], sems[_NBUF:]
    cid = lax.axis_index("c")
    sid = lax.axis_index("s")

    pltpu.sync_copy(comb.at[cid, sid], comb_v)
    pltpu.sync_copy(head.at[sid], head_v)

    def _zrow(i, c):
        for j in range(_HC // 16):
            rows_v[0, i, pl.ds(j * 16, 16)] = jnp.zeros((16,), jnp.float32)
        return c
    lax.fori_loop(0, _K, _zrow, 0)
    for t in range(_ROWS_PER_TILE // _K):
        pltpu.sync_copy(
            rows_v.at[0],
            acc.at[pl.ds(sid * _ROWS_PER_TILE + t * _K, _K)])
    plsc.subcore_barrier()

    for b in range(_PRE):
        _gather(table, comb_v, rows_v, gsems, b, b)
    for j in range(_NBUF - _PRE):
        pb = _PRE + j
        _gather(table, comb_v, rows_v, gsems, pb, pb)
        _gwait(table, comb_v, rows_v, gsems, j, j)
        _scatter(acc, head_v, rows_v, ssems, j, j)

    def _step(i, c):
        g0 = (_NBUF - _PRE) + i * _NBUF
        for j in range(_NBUF):
            g = g0 + j
            b = (_NBUF - _PRE + j) % _NBUF
            pb = j
            _swait(acc, head_v, rows_v, ssems, g + _PRE - _NBUF, pb)
            _gather(table, comb_v, rows_v, gsems, g + _PRE, pb)
            _gwait(table, comb_v, rows_v, gsems, g, b)
            _scatter(acc, head_v, rows_v, ssems, g, b)
        return c
    lax.fori_loop(0, (_G - _NBUF) // _NBUF, _step, 0)

    for j in range(_PRE):
        g = _G - _PRE + j
        b = g % _NBUF
        _gwait(table, comb_v, rows_v, gsems, g, b)
        _scatter(acc, head_v, rows_v, ssems, g, b)
    for b in range(_NBUF):
        gg = _G - _NBUF + b
        _swait(acc, head_v, rows_v, ssems, gg, b)

    plsc.subcore_barrier()
    pltpu.sync_copy(
        acc.at[pl.ds(sid * _ROWS_PER_TILE, _ROWS_PER_TILE)],
        out.at[cid, pl.ds(sid * _ROWS_PER_TILE, _ROWS_PER_TILE)])


_hop_cache = []


def _hop(table, comb4, head3):
    if not _hop_cache:
        _hop_cache.append(pl.kernel(
            _hop_body,
            out_type=jax.ShapeDtypeStruct((_NC, _ACC_ROWS, _HC), jnp.float32),
            mesh=plsc.VectorSubcoreMesh(core_axis_name="c",
                                        subcore_axis_name="s"),
            compiler_params=pltpu.CompilerParams(use_tc_tiling_on_sc=False),
            scratch_types=(
                [pltpu.VMEM((_G, _K), jnp.int32),
                 pltpu.VMEM((_G, _K), jnp.int32),
                 pltpu.VMEM((_NBUF, _K, _HC), jnp.float32),
                 pltpu.VMEM_SHARED((_ACC_ROWS, _HC), jnp.float32)]
                + [pltpu.SemaphoreType.DMA] * (2 * _NBUF)),
        ))
    p = _hop_cache[0](table, comb4, head3)
    return p[0], p[1]


def kernel(user_emb, item_emb, entity_emb, aspect_emb, edge_index, edge_type,
           ua_interact_mat, ia_interact_mat, weight, W1, W2):
    f32 = jnp.float32
    et = edge_type.astype(jnp.int32)

    user_res = pl.pallas_call(
        _user_body,
        grid=(50,),
        in_specs=[
            pl.BlockSpec((2000, _CH), lambda b: (b, 0)),
            pl.BlockSpec((2000, 64), lambda b: (b, 0)),
            pl.BlockSpec((64, _CH), lambda b: (0, 0)),
        ],
        out_specs=pl.BlockSpec((2000, _CH), lambda b: (b, 0)),
        out_shape=jax.ShapeDtypeStruct((user_emb.shape[0], _CH), f32),
    )(user_emb, ua_interact_mat, aspect_emb)

    nrow = _N_EDGES // _K
    prow = _E_PAD // _K
    comb, headp = pl.pallas_call(
        _edges_body,
        in_specs=[
            pl.BlockSpec((2, nrow, _K), lambda: (0, 0, 0)),
            pl.BlockSpec((nrow, _K), lambda: (0, 0)),
        ],
        out_specs=[
            pl.BlockSpec((2, prow, _K), lambda: (0, 0, 0)),
            pl.BlockSpec((prow, _K), lambda: (0, 0)),
        ],
        out_shape=[
            jax.ShapeDtypeStruct((2, prow, _K), jnp.int32),
            jax.ShapeDtypeStruct((prow, _K), jnp.int32),
        ],
    )(edge_index.astype(jnp.int32).reshape(2, nrow, _K), et.reshape(nrow, _K))
    comb4 = comb.reshape(_NC, _NS, _G, _K)
    head3 = headp.reshape(_NS, _G, _K)

    table1 = _build_table(entity_emb, weight)
    p1a, p1b = _hop(table1, comb4, head3)

    table2 = pl.pallas_call(
        _scale_norm_body,
        grid=(10, 2, 10),
        in_specs=[
            pl.BlockSpec((1000, _HC), lambda b, c, r: (b, 0)),
            pl.BlockSpec((1000, _HC), lambda b, c, r: (b, 0)),
            pl.BlockSpec((10, _CH), lambda b, c, r: (0, 0)),
        ],
        out_specs=pl.BlockSpec(
            (1000, _HC), lambda b, c, r: (c * 100 + r * 10 + b, 0)),
        out_shape=jax.ShapeDtypeStruct((2 * 10 * _N_ENT, _HC), jnp.float32),
        scratch_shapes=[pltpu.VMEM((1000, _CH), jnp.float32)],
    )(p1a, p1b, weight)
    p2a, p2b = _hop(table2, comb4, head3)

    item_res = pl.pallas_call(
        _final_body,
        grid=(8,),
        in_specs=[
            pl.BlockSpec((1000, _CH), lambda b: (b, 0)),
            pl.BlockSpec((1000, _HC), lambda b: (b, 0)),
            pl.BlockSpec((1000, _HC), lambda b: (b, 0)),
            pl.BlockSpec((1000, _HC), lambda b: (b, 0)),
            pl.BlockSpec((1000, _HC), lambda b: (b, 0)),
            pl.BlockSpec((1000, _CH), lambda b: (b, 0)),
            pl.BlockSpec((1000, 64), lambda b: (b, 0)),
            pl.BlockSpec((64, _CH), lambda b: (0, 0)),
            pl.BlockSpec((1, 1), lambda b: (0, 0)),
            pl.BlockSpec((1, 1), lambda b: (0, 0)),
        ],
        out_specs=pl.BlockSpec((1000, _CH), lambda b: (b, 0)),
        out_shape=jax.ShapeDtypeStruct((_N_ITEMS, _CH), f32),
    )(entity_emb, p1a, p1b, p2a, p2b, item_emb, ia_interact_mat, aspect_emb,
      W1.reshape(1, 1), W2.reshape(1, 1))

    return (item_res, user_res)
